# baseline (speedup 1.0000x reference)
import functools

import jax
import jax.numpy as jnp
from jax import lax
from jax.experimental import pallas as pl
from jax.experimental.pallas import tpu as pltpu
from jax.experimental.pallas import tpu_sc as plsc

N_PIX = 100000
N_SUP = 10000
E = 320000
D = 128
NL = 5
RHP = 5
GAMA = 0.9
EPS = 1e-5

NC = 2
NS = 16
H = D // NC
G = H // 16
INV_C = 1.0 / (2.0 + GAMA)

CB = 128
CBP = 256
EC = 80
E_PAD = NS * EC * CBP
IDXR = CBP * G // 128

PC = 50
P_PAD = NS * PC * CB
ST = 624
STX = ST * G
REM_BASE = NS * ST
REM = N_SUP - REM_BASE
RJ = 48
RJX = RJ * G
NRJ = ST // RJ
NSUPX = N_SUP * G


@functools.cache
def _mesh():
    return plsc.VectorSubcoreMesh(core_axis_name="c", subcore_axis_name="s",
                                  num_cores=NC, num_subcores=NS)


_SC_PARAMS = dict(
    compiler_params=pltpu.CompilerParams(use_tc_tiling_on_sc=False),
)


def _zero_rows(buf, n):
    @pl.loop(0, n)
    def _z(i):
        buf[i] = jnp.zeros((16,), jnp.float32)


def _zero_stripe_x(acc, zbuf, s, nsub, zn=512):
    r0 = s * nsub
    off = 0
    while off < nsub:
        n = min(zn, nsub - off)
        pltpu.sync_copy(zbuf.at[pl.ds(0, n)], acc.at[pl.ds(r0 + off, n)])
        off += n

    @pl.when(s == 0)
    def _():
        pltpu.sync_copy(zbuf.at[pl.ds(0, REM * (nsub // ST if nsub >= ST else 1))],
                        acc.at[pl.ds(NS * nsub, REM * (nsub // ST if nsub >= ST else 1))])


def _scale_rows(rowbuf, wrow, n_edges, qrowbuf=None, zrow=None):
    zv = zrow[pl.ds(0, 16)] if zrow is not None else None

    @pl.loop(0, n_edges // 16, unroll=8)
    def _blk(b8):
        wv = wrow[pl.ds(b8 * 16, 16)]
        for j in range(16):
            w = wv.at[jnp.full((16,), j, jnp.int32)].get(
                mode="promise_in_bounds")
            if qrowbuf is not None:
                qrowbuf[b8 * 16 + j] = w + zv
            for g in range(G):
                r = b8 * 16 * G + j * G + g
                rowbuf[r] = rowbuf[r] * w


def _prop_body(update, zx, sd6, ew3, sarg, out, acc,
               sdA, sdB, wrA, wrB, rowA, rowB, ubuf, sbuf,
               gsemA, gsemB, ssemA, ssemB):
    c = lax.axis_index("c")
    s = lax.axis_index("s")
    _zero_rows(rowA, G * CB)
    _zero_stripe_x(acc, rowA, s, STX, zn=G * CB)
    plsc.subcore_barrier()

    def stage_fire(ci, sd, wr, row, gsem):
        pltpu.sync_copy(sd6.at[c, s, ci], sd)
        pltpu.sync_copy(ew3.at[s, ci], wr)
        for r in range(IDXR):
            pltpu.async_copy(zx.at[sd.at[r]],
                             row.at[pl.ds(r * 128, 128)], gsem)

    def drain_gather(sd, row, gsem):
        for r in range(IDXR):
            pltpu.make_async_copy(zx.at[sd.at[r]],
                                  row.at[pl.ds(r * 128, 128)], gsem).wait()

    def process(sd, wr, row, ssem):
        _scale_rows(row, wr, CBP)
        for r in range(IDXR):
            pltpu.async_copy(row.at[pl.ds(r * 128, 128)],
                             acc.at[sd.at[IDXR + r]], ssem, add=True)

    def drain_scatter(sd, row, ssem):
        for r in range(IDXR):
            pltpu.make_async_copy(row.at[pl.ds(r * 128, 128)],
                                  acc.at[sd.at[IDXR + r]], ssem).wait()

    stage_fire(0, sdA, wrA, rowA, gsemA)

    @pl.loop(0, EC // 2)
    def _chunks(ci2):
        c0 = 2 * ci2

        @pl.when(ci2 > 0)
        def _():
            drain_scatter(sdB, rowB, ssemB)

        stage_fire(c0 + 1, sdB, wrB, rowB, gsemB)
        drain_gather(sdA, rowA, gsemA)
        process(sdA, wrA, rowA, ssemA)
        drain_gather(sdB, rowB, gsemB)
        process(sdB, wrB, rowB, ssemB)
        drain_scatter(sdA, rowA, ssemA)

        @pl.when(c0 + 2 < EC)
        def _():
            stage_fire(c0 + 2, sdA, wrA, rowA, gsemA)

    drain_scatter(sdB, rowB, ssemB)
    plsc.subcore_barrier()
    r0x = s * STX
    if not update:
        pltpu.sync_copy(acc.at[pl.ds(r0x, STX)],
                        out.at[pl.ds(c * NSUPX + r0x, STX)])

        @pl.when(s == 0)
        def _rem0():
            pltpu.sync_copy(acc.at[pl.ds(NS * STX, REM * G)],
                            out.at[pl.ds(c * NSUPX + NS * STX, REM * G)])
    else:
        def _update_rows(base, n):
            pltpu.sync_copy(acc.at[pl.ds(base, n)], ubuf.at[pl.ds(0, n)])
            pltpu.sync_copy(sarg.at[pl.ds(c * NSUPX + base, n)],
                            sbuf.at[pl.ds(0, n)])

            @pl.loop(0, n, unroll=8)
            def _r(i):
                ubuf[i] = (ubuf[i] + sbuf[i]) * INV_C
            pltpu.sync_copy(ubuf.at[pl.ds(0, n)],
                            out.at[pl.ds(c * NSUPX + base, n)])

        for j in range(4):
            _update_rows(r0x + j * (STX // 4), STX // 4)

        @pl.when(s == 0)
        def _rem():
            _update_rows(NS * STX, REM * G)


def _prop_plain_body(zx, sd6, ew3, out, acc, sdA, sdB, wrA, wrB, rowA, rowB,
                     ubuf, sbuf, gsemA, gsemB, ssemA, ssemB):
    _prop_body(False, zx, sd6, ew3, None, out, acc,
               sdA, sdB, wrA, wrB, rowA, rowB, ubuf, sbuf,
               gsemA, gsemB, ssemA, ssemB)


def _prop_update_body(zx, sd6, ew3, sarg, out, acc, sdA, sdB, wrA, wrB, rowA,
                      rowB, ubuf, sbuf, gsemA, gsemB, ssemA, ssemB):
    _prop_body(True, zx, sd6, ew3, sarg, out, acc,
               sdA, sdB, wrA, wrB, rowA, rowB, ubuf, sbuf,
               gsemA, gsemB, ssemA, ssemB)


@functools.cache
def _make_prop(update):
    return pl.kernel(
        _prop_update_body if update else _prop_plain_body,
        out_type=jax.ShapeDtypeStruct((NC * NSUPX, 16), jnp.float32),
        mesh=_mesh(),
        scratch_types=[
            pltpu.VMEM_SHARED((NSUPX, 16), jnp.float32),
            pltpu.VMEM((2 * IDXR, 128), jnp.int32),
            pltpu.VMEM((2 * IDXR, 128), jnp.int32),
            pltpu.VMEM((CBP,), jnp.float32),
            pltpu.VMEM((CBP,), jnp.float32),
            pltpu.VMEM((G * CBP, 16), jnp.float32),
            pltpu.VMEM((G * CBP, 16), jnp.float32),
            pltpu.VMEM((STX // 4, 16), jnp.float32),
            pltpu.VMEM((STX // 4, 16), jnp.float32),
            pltpu.SemaphoreType.DMA,
            pltpu.SemaphoreType.DMA,
            pltpu.SemaphoreType.DMA,
            pltpu.SemaphoreType.DMA,
        ],
        **_SC_PARAMS,
    )


def _prop_plain(*args):
    return _make_prop(False)(*args)


def _prop_update(*args):
    return _make_prop(True)(*args)


def _pool_body(xflat, pix3, px5, q3, hsum, cnt, acc, cacc,
               pixbuf, pxbuf, wrow, rowbuf, qrowbuf, zrow, sem):
    c = lax.axis_index("c")
    s = lax.axis_index("s")
    pltpu.sync_copy(pix3.at[s], pixbuf)
    _zero_rows(rowbuf, G * CB)
    _zero_rows(qrowbuf, CB)
    zrow[pl.ds(0, 16)] = jnp.zeros((16,), jnp.float32)
    _zero_stripe_x(acc, rowbuf, s, STX)
    _zero_stripe_x(cacc, qrowbuf, s, ST, zn=CB)
    plsc.subcore_barrier()

    @pl.loop(0, PC)
    def _chunks(ci):
        base = (c * P_PAD + s * (PC * CB) + ci * CB) * G
        pltpu.sync_copy(xflat.at[pl.ds(base, G * CB)], rowbuf)
        pltpu.sync_copy(px5.at[s, ci], pxbuf)
        pltpu.sync_copy(q3.at[s, ci], wrow)
        _scale_rows(rowbuf, wrow, CB, qrowbuf=qrowbuf, zrow=zrow)
        for g in range(G):
            pltpu.sync_copy(rowbuf.at[pl.ds(g * CB, CB)],
                            acc.at[pxbuf.at[g]], add=True)
        pltpu.sync_copy(qrowbuf, cacc.at[pixbuf.at[ci]], add=True)

    plsc.subcore_barrier()
    r0x = s * STX
    pltpu.sync_copy(acc.at[pl.ds(r0x, STX)],
                    hsum.at[pl.ds(c * NSUPX + r0x, STX)])

    @pl.when(s == 0)
    def _rem0():
        pltpu.sync_copy(acc.at[pl.ds(NS * STX, REM * G)],
                        hsum.at[pl.ds(c * NSUPX + NS * STX, REM * G)])

    @pl.when(c == 0)
    def _():
        r0 = s * ST
        pltpu.sync_copy(cacc.at[pl.ds(r0, ST)], cnt.at[pl.ds(r0, ST)])

        @pl.when(s == 0)
        def _rem1():
            pltpu.sync_copy(cacc.at[pl.ds(REM_BASE, REM)],
                            cnt.at[pl.ds(REM_BASE, REM)])


@functools.cache
def _pool_kernel():
    return pl.kernel(
        _pool_body,
        out_type=(jax.ShapeDtypeStruct((NC * NSUPX, 16), jnp.float32),
                  jax.ShapeDtypeStruct((N_SUP, 16), jnp.float32)),
        mesh=_mesh(),
        scratch_types=[
            pltpu.VMEM_SHARED((NSUPX, 16), jnp.float32),
            pltpu.VMEM_SHARED((N_SUP, 16), jnp.float32),
            pltpu.VMEM((PC, CB), jnp.int32),
            pltpu.VMEM((G, CB), jnp.int32),
            pltpu.VMEM((CB,), jnp.float32),
            pltpu.VMEM((G * CB, 16), jnp.float32),
            pltpu.VMEM((CB, 16), jnp.float32),
            pltpu.VMEM((16,), jnp.float32),
            pltpu.SemaphoreType.DMA,
        ],
        **_SC_PARAMS,
    )


def _pool(*args):
    return _pool_kernel()(*args)


def _final_body(hx, pg6, q3, out0, out1, pgbuf, wrow, rowbuf, sem):
    c = lax.axis_index("c")
    s = lax.axis_index("s")
    nfull = jnp.where(s == NS - 1, 31, PC)

    def _do_chunk(ci, n):
        pltpu.sync_copy(pg6.at[c, s, ci], pgbuf)
        pltpu.sync_copy(q3.at[s, ci], wrow)
        for g in range(G):
            pltpu.async_copy(hx.at[pgbuf.at[g]],
                             rowbuf.at[pl.ds(g * CB, CB)], sem).wait()
        _scale_rows(rowbuf, wrow, CB)
        gbase = (s * (PC * CB) + ci * CB) * G
        for co, outr in ((0, out0), (1, out1)):
            @pl.when(c == co)
            def _():
                pltpu.sync_copy(rowbuf.at[pl.ds(0, n * G)],
                                outr.at[pl.ds(gbase, n * G)])

    @pl.loop(0, nfull)
    def _chunks(ci):
        _do_chunk(ci, CB)

    @pl.when(s == NS - 1)
    def _tail():
        _do_chunk(31, 32)


@functools.cache
def _final_kernel():
    return pl.kernel(
        _final_body,
        out_type=(jax.ShapeDtypeStruct((N_PIX * G, 16), jnp.float32),
                  jax.ShapeDtypeStruct((N_PIX * G, 16), jnp.float32)),
        mesh=_mesh(),
        scratch_types=[
            pltpu.VMEM((G, CB), jnp.int32),
            pltpu.VMEM((CB,), jnp.float32),
            pltpu.VMEM((G * CB, 16), jnp.float32),
            pltpu.SemaphoreType.DMA,
        ],
        **_SC_PARAMS,
    )


def _final(*args):
    return _final_kernel()(*args)


def _cat(flat):
    return jnp.concatenate([flat[:N_SUP], flat[N_SUP:]], axis=1)


def _split_write(out_ref, val):
    out_ref[0:N_SUP, :] = val[:, :H]
    out_ref[N_SUP:2 * N_SUP, :] = val[:, H:]


def _combine_mm_body(hsum_ref, cnt_ref, w_ref, b_ref, h_ref, y_ref):
    cntc = cnt_ref[:, 0:1]
    inv = jnp.where(cntc > 0, 1.0 / cntc, 0.0)
    h = _cat(hsum_ref[...]) * inv
    y = jnp.dot(h, w_ref[...], preferred_element_type=jnp.float32,
                precision=lax.Precision.HIGHEST) + b_ref[...][None, :]
    _split_write(h_ref, h)
    _split_write(y_ref, y)


_combine_mm = pl.pallas_call(
    _combine_mm_body,
    out_shape=(jax.ShapeDtypeStruct((NC * N_SUP, H), jnp.float32),
               jax.ShapeDtypeStruct((NC * N_SUP, H), jnp.float32)),
)


def _bn(x, gamma, beta):
    mean = jnp.mean(x, axis=0, keepdims=True)
    var = jnp.mean((x - mean) ** 2, axis=0, keepdims=True)
    xn = (x - mean) * lax.rsqrt(var + EPS) * gamma[None, :] + beta[None, :]
    return jnp.where(xn >= 0, xn, 0.01 * xn)


def _bn_mm_body(x_ref, g_ref, be_ref, w_ref, b_ref, h_ref, y_ref):
    h = _bn(_cat(x_ref[...]), g_ref[...], be_ref[...])
    y = jnp.dot(h, w_ref[...], preferred_element_type=jnp.float32,
                precision=lax.Precision.HIGHEST) + b_ref[...][None, :]
    _split_write(h_ref, h)
    _split_write(y_ref, y)


_bn_mm = pl.pallas_call(
    _bn_mm_body,
    out_shape=(jax.ShapeDtypeStruct((NC * N_SUP, H), jnp.float32),
               jax.ShapeDtypeStruct((NC * N_SUP, H), jnp.float32)),
)


def _bn_only_body(x_ref, g_ref, be_ref, h_ref):
    _split_write(h_ref, _bn(_cat(x_ref[...]), g_ref[...], be_ref[...]))


_bn_only = pl.pallas_call(
    _bn_only_body,
    out_shape=jax.ShapeDtypeStruct((NC * N_SUP, H), jnp.float32),
)


def _expand4(idx, core_offset):
    base = (idx + core_offset) * G
    return jnp.reshape(base[:, None] + jnp.arange(G, dtype=jnp.int32), (-1,))


def _to_x(a2d):
    return a2d.reshape(-1, 16)


def _from_x(ax):
    return ax.reshape(-1, H)


def kernel(x, pix2sup, edge_index, edge_value, q_values, W, b, gamma, beta):
    src = edge_index[0].astype(jnp.int32)
    dst = edge_index[1].astype(jnp.int32)
    ew = edge_value.astype(jnp.float32)

    pad_e = E_PAD - E
    spread_e = (jnp.arange(pad_e, dtype=jnp.int32) * 7) % N_SUP
    srcp = jnp.concatenate([src, spread_e])
    dstp = jnp.concatenate([dst, spread_e])
    ewp = jnp.concatenate([ew, jnp.zeros((pad_e,), jnp.float32)])
    dx = _expand4(dstp, 0).reshape(NS, EC, IDXR, 128)
    sde = []
    for c in range(NC):
        sx = _expand4(srcp, c * N_SUP).reshape(NS, EC, IDXR, 128)
        sde.append(jnp.concatenate([sx, dx], axis=2))
    sd6 = jnp.stack(sde)
    ew3 = ewp.reshape(NS, EC, CBP)

    pad_p = P_PAD - N_PIX
    spread_p = (jnp.arange(pad_p, dtype=jnp.int32) * 11) % N_SUP
    pixp = jnp.concatenate([pix2sup.astype(jnp.int32), spread_p])
    qp = jnp.concatenate([q_values.astype(jnp.float32),
                          jnp.zeros((pad_p,), jnp.float32)])
    pix3 = pixp.reshape(NS, PC, CB)
    px5 = _expand4(pixp, 0).reshape(NS, PC, G, CB)
    pg6 = jnp.stack([_expand4(pixp, c * N_SUP).reshape(NS, PC, G, CB)
                     for c in range(NC)])
    q3 = qp.reshape(NS, PC, CB)

    xh = x.reshape(N_PIX, NC, H).transpose(1, 0, 2)
    xflat = _to_x(jnp.concatenate(
        [xh, jnp.zeros((NC, pad_p, H), jnp.float32)], axis=1).reshape(-1, H))

    hsumx, cnt = _pool(xflat, pix3, px5, q3)
    h, y = _combine_mm(_from_x(hsumx), cnt, W[0], b[0])
    for i in range(NL):
        xs = _prop_plain(_to_x(y), sd6, ew3)
        z = _to_x(h)
        for _ in range(RHP):
            z = _prop_update(z, sd6, ew3, xs)
        if i < NL - 1:
            h, y = _bn_mm(_from_x(z), gamma[i], beta[i], W[i + 1], b[i + 1])
        else:
            h = _bn_only(_from_x(z), gamma[i], beta[i])
    out0, out1 = _final(_to_x(h), pg6, q3)
    return jnp.concatenate([_from_x(out0), _from_x(out1)], axis=1)


# speedup vs baseline: 1.0121x; 1.0121x over previous
import functools

import jax
import jax.numpy as jnp
from jax import lax
from jax.experimental import pallas as pl
from jax.experimental.pallas import tpu as pltpu
from jax.experimental.pallas import tpu_sc as plsc

N_PIX = 100000
N_SUP = 10000
E = 320000
D = 128
NL = 5
RHP = 5
GAMA = 0.9
EPS = 1e-5

NC = 2
NS = 16
H = D // NC
G = H // 16
INV_C = 1.0 / (2.0 + GAMA)

CB = 128
CBP = 256
EC = 80
E_PAD = NS * EC * CBP
IDXR = CBP * G // 128

PC = 50
P_PAD = NS * PC * CB
ST = 624
STX = ST * G
REM_BASE = NS * ST
REM = N_SUP - REM_BASE
RJ = 48
RJX = RJ * G
NRJ = ST // RJ
NSUPX = N_SUP * G


@functools.cache
def _mesh():
    return plsc.VectorSubcoreMesh(core_axis_name="c", subcore_axis_name="s",
                                  num_cores=NC, num_subcores=NS)


_SC_PARAMS = dict(
    compiler_params=pltpu.CompilerParams(use_tc_tiling_on_sc=False),
)


def _zero_rows(buf, n):
    @pl.loop(0, n)
    def _z(i):
        buf[i] = jnp.zeros((16,), jnp.float32)


def _zero_stripe_x(acc, zbuf, s, nsub, zn=512):
    r0 = s * nsub
    off = 0
    while off < nsub:
        n = min(zn, nsub - off)
        pltpu.sync_copy(zbuf.at[pl.ds(0, n)], acc.at[pl.ds(r0 + off, n)])
        off += n

    @pl.when(s == 0)
    def _():
        pltpu.sync_copy(zbuf.at[pl.ds(0, REM * (nsub // ST if nsub >= ST else 1))],
                        acc.at[pl.ds(NS * nsub, REM * (nsub // ST if nsub >= ST else 1))])


def _scale_rows(rowbuf, wrow, n_edges, qrowbuf=None, zrow=None):
    zv = zrow[pl.ds(0, 16)] if zrow is not None else None

    @pl.loop(0, n_edges // 16, unroll=4)
    def _blk(b8):
        wv = wrow[pl.ds(b8 * 16, 16)]
        for j in range(16):
            w = wv.at[jnp.full((16,), j, jnp.int32)].get(
                mode="promise_in_bounds")
            if qrowbuf is not None:
                qrowbuf[b8 * 16 + j] = w + zv
            for g in range(G):
                r = b8 * 16 * G + j * G + g
                rowbuf[r] = rowbuf[r] * w


def _prop_body(update, zx, sd6, ew3, sarg, out, acc,
               sdA, sdB, wrA, wrB, rowA, rowB, ubuf, sbuf,
               gsemA, gsemB, ssemA, ssemB):
    c = lax.axis_index("c")
    s = lax.axis_index("s")
    _zero_rows(rowA, G * CB)
    _zero_stripe_x(acc, rowA, s, STX, zn=G * CB)
    plsc.subcore_barrier()

    def stage_fire(ci, sd, wr, row, gsem):
        pltpu.sync_copy(sd6.at[c, s, ci], sd)
        pltpu.sync_copy(ew3.at[s, ci], wr)
        for r in range(IDXR):
            pltpu.async_copy(zx.at[sd.at[r]],
                             row.at[pl.ds(r * 128, 128)], gsem)

    def drain_gather(sd, row, gsem):
        for r in range(IDXR):
            pltpu.make_async_copy(zx.at[sd.at[r]],
                                  row.at[pl.ds(r * 128, 128)], gsem).wait()

    def process(sd, wr, row, ssem):
        _scale_rows(row, wr, CBP)
        for r in range(IDXR):
            pltpu.async_copy(row.at[pl.ds(r * 128, 128)],
                             acc.at[sd.at[IDXR + r]], ssem, add=True)

    def drain_scatter(sd, row, ssem):
        for r in range(IDXR):
            pltpu.make_async_copy(row.at[pl.ds(r * 128, 128)],
                                  acc.at[sd.at[IDXR + r]], ssem).wait()

    stage_fire(0, sdA, wrA, rowA, gsemA)

    @pl.loop(0, EC // 2)
    def _chunks(ci2):
        c0 = 2 * ci2

        @pl.when(ci2 > 0)
        def _():
            drain_scatter(sdB, rowB, ssemB)

        stage_fire(c0 + 1, sdB, wrB, rowB, gsemB)
        drain_gather(sdA, rowA, gsemA)
        process(sdA, wrA, rowA, ssemA)
        drain_gather(sdB, rowB, gsemB)
        process(sdB, wrB, rowB, ssemB)
        drain_scatter(sdA, rowA, ssemA)

        @pl.when(c0 + 2 < EC)
        def _():
            stage_fire(c0 + 2, sdA, wrA, rowA, gsemA)

    drain_scatter(sdB, rowB, ssemB)
    plsc.subcore_barrier()
    r0x = s * STX
    if not update:
        pltpu.sync_copy(acc.at[pl.ds(r0x, STX)],
                        out.at[pl.ds(c * NSUPX + r0x, STX)])

        @pl.when(s == 0)
        def _rem0():
            pltpu.sync_copy(acc.at[pl.ds(NS * STX, REM * G)],
                            out.at[pl.ds(c * NSUPX + NS * STX, REM * G)])
    else:
        def _update_rows(base, n):
            pltpu.sync_copy(acc.at[pl.ds(base, n)], ubuf.at[pl.ds(0, n)])
            pltpu.sync_copy(sarg.at[pl.ds(c * NSUPX + base, n)],
                            sbuf.at[pl.ds(0, n)])

            @pl.loop(0, n, unroll=8)
            def _r(i):
                ubuf[i] = (ubuf[i] + sbuf[i]) * INV_C
            pltpu.sync_copy(ubuf.at[pl.ds(0, n)],
                            out.at[pl.ds(c * NSUPX + base, n)])

        for j in range(4):
            _update_rows(r0x + j * (STX // 4), STX // 4)

        @pl.when(s == 0)
        def _rem():
            _update_rows(NS * STX, REM * G)


def _prop_plain_body(zx, sd6, ew3, out, acc, sdA, sdB, wrA, wrB, rowA, rowB,
                     ubuf, sbuf, gsemA, gsemB, ssemA, ssemB):
    _prop_body(False, zx, sd6, ew3, None, out, acc,
               sdA, sdB, wrA, wrB, rowA, rowB, ubuf, sbuf,
               gsemA, gsemB, ssemA, ssemB)


def _prop_update_body(zx, sd6, ew3, sarg, out, acc, sdA, sdB, wrA, wrB, rowA,
                      rowB, ubuf, sbuf, gsemA, gsemB, ssemA, ssemB):
    _prop_body(True, zx, sd6, ew3, sarg, out, acc,
               sdA, sdB, wrA, wrB, rowA, rowB, ubuf, sbuf,
               gsemA, gsemB, ssemA, ssemB)


@functools.cache
def _make_prop(update):
    return pl.kernel(
        _prop_update_body if update else _prop_plain_body,
        out_type=jax.ShapeDtypeStruct((NC * NSUPX, 16), jnp.float32),
        mesh=_mesh(),
        scratch_types=[
            pltpu.VMEM_SHARED((NSUPX, 16), jnp.float32),
            pltpu.VMEM((2 * IDXR, 128), jnp.int32),
            pltpu.VMEM((2 * IDXR, 128), jnp.int32),
            pltpu.VMEM((CBP,), jnp.float32),
            pltpu.VMEM((CBP,), jnp.float32),
            pltpu.VMEM((G * CBP, 16), jnp.float32),
            pltpu.VMEM((G * CBP, 16), jnp.float32),
            pltpu.VMEM((STX // 4, 16), jnp.float32),
            pltpu.VMEM((STX // 4, 16), jnp.float32),
            pltpu.SemaphoreType.DMA,
            pltpu.SemaphoreType.DMA,
            pltpu.SemaphoreType.DMA,
            pltpu.SemaphoreType.DMA,
        ],
        **_SC_PARAMS,
    )


def _prop_plain(*args):
    return _make_prop(False)(*args)


def _prop_update(*args):
    return _make_prop(True)(*args)


def _pool_body(xflat, pix3, px5, q3, hsum, cnt, acc, cacc,
               pixbuf, pxbuf, wrow, rowbuf, qrowbuf, zrow, sem):
    c = lax.axis_index("c")
    s = lax.axis_index("s")
    pltpu.sync_copy(pix3.at[s], pixbuf)
    _zero_rows(rowbuf, G * CB)
    _zero_rows(qrowbuf, CB)
    zrow[pl.ds(0, 16)] = jnp.zeros((16,), jnp.float32)
    _zero_stripe_x(acc, rowbuf, s, STX)
    _zero_stripe_x(cacc, qrowbuf, s, ST, zn=CB)
    plsc.subcore_barrier()

    @pl.loop(0, PC)
    def _chunks(ci):
        base = (c * P_PAD + s * (PC * CB) + ci * CB) * G
        pltpu.sync_copy(xflat.at[pl.ds(base, G * CB)], rowbuf)
        pltpu.sync_copy(px5.at[s, ci], pxbuf)
        pltpu.sync_copy(q3.at[s, ci], wrow)
        _scale_rows(rowbuf, wrow, CB, qrowbuf=qrowbuf, zrow=zrow)
        for g in range(G):
            pltpu.sync_copy(rowbuf.at[pl.ds(g * CB, CB)],
                            acc.at[pxbuf.at[g]], add=True)
        pltpu.sync_copy(qrowbuf, cacc.at[pixbuf.at[ci]], add=True)

    plsc.subcore_barrier()
    r0x = s * STX
    pltpu.sync_copy(acc.at[pl.ds(r0x, STX)],
                    hsum.at[pl.ds(c * NSUPX + r0x, STX)])

    @pl.when(s == 0)
    def _rem0():
        pltpu.sync_copy(acc.at[pl.ds(NS * STX, REM * G)],
                        hsum.at[pl.ds(c * NSUPX + NS * STX, REM * G)])

    @pl.when(c == 0)
    def _():
        r0 = s * ST
        pltpu.sync_copy(cacc.at[pl.ds(r0, ST)], cnt.at[pl.ds(r0, ST)])

        @pl.when(s == 0)
        def _rem1():
            pltpu.sync_copy(cacc.at[pl.ds(REM_BASE, REM)],
                            cnt.at[pl.ds(REM_BASE, REM)])


@functools.cache
def _pool_kernel():
    return pl.kernel(
        _pool_body,
        out_type=(jax.ShapeDtypeStruct((NC * NSUPX, 16), jnp.float32),
                  jax.ShapeDtypeStruct((N_SUP, 16), jnp.float32)),
        mesh=_mesh(),
        scratch_types=[
            pltpu.VMEM_SHARED((NSUPX, 16), jnp.float32),
            pltpu.VMEM_SHARED((N_SUP, 16), jnp.float32),
            pltpu.VMEM((PC, CB), jnp.int32),
            pltpu.VMEM((G, CB), jnp.int32),
            pltpu.VMEM((CB,), jnp.float32),
            pltpu.VMEM((G * CB, 16), jnp.float32),
            pltpu.VMEM((CB, 16), jnp.float32),
            pltpu.VMEM((16,), jnp.float32),
            pltpu.SemaphoreType.DMA,
        ],
        **_SC_PARAMS,
    )


def _pool(*args):
    return _pool_kernel()(*args)


def _final_body(hx, pg6, q3, out0, out1, pgbuf, wrow, rowbuf, sem):
    c = lax.axis_index("c")
    s = lax.axis_index("s")
    nfull = jnp.where(s == NS - 1, 31, PC)

    def _do_chunk(ci, n):
        pltpu.sync_copy(pg6.at[c, s, ci], pgbuf)
        pltpu.sync_copy(q3.at[s, ci], wrow)
        for g in range(G):
            pltpu.async_copy(hx.at[pgbuf.at[g]],
                             rowbuf.at[pl.ds(g * CB, CB)], sem).wait()
        _scale_rows(rowbuf, wrow, CB)
        gbase = (s * (PC * CB) + ci * CB) * G
        for co, outr in ((0, out0), (1, out1)):
            @pl.when(c == co)
            def _():
                pltpu.sync_copy(rowbuf.at[pl.ds(0, n * G)],
                                outr.at[pl.ds(gbase, n * G)])

    @pl.loop(0, nfull)
    def _chunks(ci):
        _do_chunk(ci, CB)

    @pl.when(s == NS - 1)
    def _tail():
        _do_chunk(31, 32)


@functools.cache
def _final_kernel():
    return pl.kernel(
        _final_body,
        out_type=(jax.ShapeDtypeStruct((N_PIX * G, 16), jnp.float32),
                  jax.ShapeDtypeStruct((N_PIX * G, 16), jnp.float32)),
        mesh=_mesh(),
        scratch_types=[
            pltpu.VMEM((G, CB), jnp.int32),
            pltpu.VMEM((CB,), jnp.float32),
            pltpu.VMEM((G * CB, 16), jnp.float32),
            pltpu.SemaphoreType.DMA,
        ],
        **_SC_PARAMS,
    )


def _final(*args):
    return _final_kernel()(*args)


def _cat(flat):
    return jnp.concatenate([flat[:N_SUP], flat[N_SUP:]], axis=1)


def _split_write(out_ref, val):
    out_ref[0:N_SUP, :] = val[:, :H]
    out_ref[N_SUP:2 * N_SUP, :] = val[:, H:]


def _combine_mm_body(hsum_ref, cnt_ref, w_ref, b_ref, h_ref, y_ref):
    cntc = cnt_ref[:, 0:1]
    inv = jnp.where(cntc > 0, 1.0 / cntc, 0.0)
    h = _cat(hsum_ref[...]) * inv
    y = jnp.dot(h, w_ref[...], preferred_element_type=jnp.float32,
                precision=lax.Precision.HIGHEST) + b_ref[...][None, :]
    _split_write(h_ref, h)
    _split_write(y_ref, y)


_combine_mm = pl.pallas_call(
    _combine_mm_body,
    out_shape=(jax.ShapeDtypeStruct((NC * N_SUP, H), jnp.float32),
               jax.ShapeDtypeStruct((NC * N_SUP, H), jnp.float32)),
)


def _bn(x, gamma, beta):
    mean = jnp.mean(x, axis=0, keepdims=True)
    var = jnp.mean((x - mean) ** 2, axis=0, keepdims=True)
    xn = (x - mean) * lax.rsqrt(var + EPS) * gamma[None, :] + beta[None, :]
    return jnp.where(xn >= 0, xn, 0.01 * xn)


def _bn_mm_body(x_ref, g_ref, be_ref, w_ref, b_ref, h_ref, y_ref):
    h = _bn(_cat(x_ref[...]), g_ref[...], be_ref[...])
    y = jnp.dot(h, w_ref[...], preferred_element_type=jnp.float32,
                precision=lax.Precision.HIGHEST) + b_ref[...][None, :]
    _split_write(h_ref, h)
    _split_write(y_ref, y)


_bn_mm = pl.pallas_call(
    _bn_mm_body,
    out_shape=(jax.ShapeDtypeStruct((NC * N_SUP, H), jnp.float32),
               jax.ShapeDtypeStruct((NC * N_SUP, H), jnp.float32)),
)


def _bn_only_body(x_ref, g_ref, be_ref, h_ref):
    _split_write(h_ref, _bn(_cat(x_ref[...]), g_ref[...], be_ref[...]))


_bn_only = pl.pallas_call(
    _bn_only_body,
    out_shape=jax.ShapeDtypeStruct((NC * N_SUP, H), jnp.float32),
)


def _expand4(idx, core_offset):
    base = (idx + core_offset) * G
    return jnp.reshape(base[:, None] + jnp.arange(G, dtype=jnp.int32), (-1,))


def _to_x(a2d):
    return a2d.reshape(-1, 16)


def _from_x(ax):
    return ax.reshape(-1, H)


def kernel(x, pix2sup, edge_index, edge_value, q_values, W, b, gamma, beta):
    src = edge_index[0].astype(jnp.int32)
    dst = edge_index[1].astype(jnp.int32)
    ew = edge_value.astype(jnp.float32)

    pad_e = E_PAD - E
    spread_e = (jnp.arange(pad_e, dtype=jnp.int32) * 7) % N_SUP
    srcp = jnp.concatenate([src, spread_e])
    dstp = jnp.concatenate([dst, spread_e])
    ewp = jnp.concatenate([ew, jnp.zeros((pad_e,), jnp.float32)])
    dx = _expand4(dstp, 0).reshape(NS, EC, IDXR, 128)
    sde = []
    for c in range(NC):
        sx = _expand4(srcp, c * N_SUP).reshape(NS, EC, IDXR, 128)
        sde.append(jnp.concatenate([sx, dx], axis=2))
    sd6 = jnp.stack(sde)
    ew3 = ewp.reshape(NS, EC, CBP)

    pad_p = P_PAD - N_PIX
    spread_p = (jnp.arange(pad_p, dtype=jnp.int32) * 11) % N_SUP
    pixp = jnp.concatenate([pix2sup.astype(jnp.int32), spread_p])
    qp = jnp.concatenate([q_values.astype(jnp.float32),
                          jnp.zeros((pad_p,), jnp.float32)])
    pix3 = pixp.reshape(NS, PC, CB)
    px5 = _expand4(pixp, 0).reshape(NS, PC, G, CB)
    pg6 = jnp.stack([_expand4(pixp, c * N_SUP).reshape(NS, PC, G, CB)
                     for c in range(NC)])
    q3 = qp.reshape(NS, PC, CB)

    xh = x.reshape(N_PIX, NC, H).transpose(1, 0, 2)
    xflat = _to_x(jnp.concatenate(
        [xh, jnp.zeros((NC, pad_p, H), jnp.float32)], axis=1).reshape(-1, H))

    hsumx, cnt = _pool(xflat, pix3, px5, q3)
    h, y = _combine_mm(_from_x(hsumx), cnt, W[0], b[0])
    for i in range(NL):
        xs = _prop_plain(_to_x(y), sd6, ew3)
        z = _to_x(h)
        for _ in range(RHP):
            z = _prop_update(z, sd6, ew3, xs)
        if i < NL - 1:
            h, y = _bn_mm(_from_x(z), gamma[i], beta[i], W[i + 1], b[i + 1])
        else:
            h = _bn_only(_from_x(z), gamma[i], beta[i])
    out0, out1 = _final(_to_x(h), pg6, q3)
    return jnp.concatenate([_from_x(out0), _from_x(out1)], axis=1)


# speedup vs baseline: 1.2304x; 1.2157x over previous
import functools

import jax
import jax.numpy as jnp
from jax import lax
from jax.experimental import pallas as pl
from jax.experimental.pallas import tpu as pltpu
from jax.experimental.pallas import tpu_sc as plsc

N_PIX = 100000
N_SUP = 10000
E = 320000
D = 128
NL = 5
RHP = 5
GAMA = 0.9
EPS = 1e-5

NC = 2
NS = 16
H = D // NC
G = H // 16
INV_C = 1.0 / (2.0 + GAMA)

CB = 128
CBP = 256
EC = 80
E_PAD = NS * EC * CBP
IDXR = CBP * G // 128

PC = 50
P_PAD = NS * PC * CB
ST = 624
STX = ST * G
REM_BASE = NS * ST
REM = N_SUP - REM_BASE
RJ = 48
RJX = RJ * G
NRJ = ST // RJ
NSUPX = N_SUP * G


@functools.cache
def _mesh():
    return plsc.VectorSubcoreMesh(core_axis_name="c", subcore_axis_name="s",
                                  num_cores=NC, num_subcores=NS)


_SC_PARAMS = dict(
    compiler_params=pltpu.CompilerParams(use_tc_tiling_on_sc=False),
)


def _zero_rows(buf, n):
    @pl.loop(0, n)
    def _z(i):
        buf[i] = jnp.zeros((16,), jnp.float32)


def _zero_stripe_x(acc, zbuf, s, nsub, zn=512):
    r0 = s * nsub
    off = 0
    while off < nsub:
        n = min(zn, nsub - off)
        pltpu.sync_copy(zbuf.at[pl.ds(0, n)], acc.at[pl.ds(r0 + off, n)])
        off += n

    @pl.when(s == 0)
    def _():
        pltpu.sync_copy(zbuf.at[pl.ds(0, REM * (nsub // ST if nsub >= ST else 1))],
                        acc.at[pl.ds(NS * nsub, REM * (nsub // ST if nsub >= ST else 1))])


def _scale_rows(rowbuf, wrow, n_edges, qrowbuf=None, zrow=None):
    zv = zrow[pl.ds(0, 16)] if zrow is not None else None

    @pl.loop(0, n_edges // 16, unroll=4)
    def _blk(b8):
        wv = wrow[pl.ds(b8 * 16, 16)]
        for j in range(16):
            w = wv.at[jnp.full((16,), j, jnp.int32)].get(
                mode="promise_in_bounds")
            if qrowbuf is not None:
                qrowbuf[b8 * 16 + j] = w + zv
            for g in range(G):
                r = b8 * 16 * G + j * G + g
                rowbuf[r] = rowbuf[r] * w


def _prop_body(zx, sd6, ew3, sarg, out, acc,
               sdA, sdB, wrA, wrB, rowA, rowB, ubuf, sbuf,
               gsemA, gsemB, ssemA, ssemB):
    c = lax.axis_index("c")
    s = lax.axis_index("s")
    _zero_rows(rowA, G * CB)
    _zero_stripe_x(acc, rowA, s, STX, zn=G * CB)
    plsc.subcore_barrier()

    def stage_fire(ci, sd, wr, row, gsem):
        d1 = pltpu.async_copy(sd6.at[c, s, ci], sd, gsem)
        d2 = pltpu.async_copy(ew3.at[s, ci], wr, gsem)
        d1.wait()
        d2.wait()
        for r in range(IDXR):
            pltpu.async_copy(zx.at[sd.at[r]],
                             row.at[pl.ds(r * 128, 128)], gsem)

    def drain_gather(sd, row, gsem):
        for r in range(IDXR):
            pltpu.make_async_copy(zx.at[sd.at[r]],
                                  row.at[pl.ds(r * 128, 128)], gsem).wait()

    def process(sd, wr, row, ssem):
        _scale_rows(row, wr, CBP)
        for r in range(IDXR):
            pltpu.async_copy(row.at[pl.ds(r * 128, 128)],
                             acc.at[sd.at[IDXR + r]], ssem, add=True)

    def drain_scatter(sd, row, ssem):
        for r in range(IDXR):
            pltpu.make_async_copy(row.at[pl.ds(r * 128, 128)],
                                  acc.at[sd.at[IDXR + r]], ssem).wait()

    stage_fire(0, sdA, wrA, rowA, gsemA)

    @pl.loop(0, EC // 2)
    def _chunks(ci2):
        c0 = 2 * ci2

        @pl.when(ci2 > 0)
        def _():
            drain_scatter(sdB, rowB, ssemB)

        stage_fire(c0 + 1, sdB, wrB, rowB, gsemB)
        drain_gather(sdA, rowA, gsemA)
        process(sdA, wrA, rowA, ssemA)
        drain_gather(sdB, rowB, gsemB)
        process(sdB, wrB, rowB, ssemB)
        drain_scatter(sdA, rowA, ssemA)

        @pl.when(c0 + 2 < EC)
        def _():
            stage_fire(c0 + 2, sdA, wrA, rowA, gsemA)

    drain_scatter(sdB, rowB, ssemB)
    plsc.subcore_barrier()
    r0x = s * STX

    def _update_rows(base, n):
        pltpu.sync_copy(acc.at[pl.ds(base, n)], ubuf.at[pl.ds(0, n)])
        pltpu.sync_copy(sarg.at[pl.ds(c * NSUPX + base, n)],
                        sbuf.at[pl.ds(0, n)])

        @pl.loop(0, n, unroll=8)
        def _r(i):
            ubuf[i] = ubuf[i] * INV_C + sbuf[i]
        pltpu.sync_copy(ubuf.at[pl.ds(0, n)],
                        out.at[pl.ds(c * NSUPX + base, n)])

    for j in range(4):
        _update_rows(r0x + j * (STX // 4), STX // 4)

    @pl.when(s == 0)
    def _rem():
        _update_rows(NS * STX, REM * G)


@functools.cache
def _make_prop():
    return pl.kernel(
        _prop_body,
        out_type=jax.ShapeDtypeStruct((NC * NSUPX, 16), jnp.float32),
        mesh=_mesh(),
        scratch_types=[
            pltpu.VMEM_SHARED((NSUPX, 16), jnp.float32),
            pltpu.VMEM((2 * IDXR, 128), jnp.int32),
            pltpu.VMEM((2 * IDXR, 128), jnp.int32),
            pltpu.VMEM((CBP,), jnp.float32),
            pltpu.VMEM((CBP,), jnp.float32),
            pltpu.VMEM((G * CBP, 16), jnp.float32),
            pltpu.VMEM((G * CBP, 16), jnp.float32),
            pltpu.VMEM((STX // 4, 16), jnp.float32),
            pltpu.VMEM((STX // 4, 16), jnp.float32),
            pltpu.SemaphoreType.DMA,
            pltpu.SemaphoreType.DMA,
            pltpu.SemaphoreType.DMA,
            pltpu.SemaphoreType.DMA,
        ],
        **_SC_PARAMS,
    )


def _prop_update(*args):
    return _make_prop()(*args)


def _pool_body(xflat, pix3, px5, q3, hsum, cnt, acc, cacc,
               pixbuf, pxbuf, wrow, rowbuf, qrowbuf, zrow, sem):
    c = lax.axis_index("c")
    s = lax.axis_index("s")
    pltpu.sync_copy(pix3.at[s], pixbuf)
    _zero_rows(rowbuf, G * CB)
    _zero_rows(qrowbuf, CB)
    zrow[pl.ds(0, 16)] = jnp.zeros((16,), jnp.float32)
    _zero_stripe_x(acc, rowbuf, s, STX)
    _zero_stripe_x(cacc, qrowbuf, s, ST, zn=CB)
    plsc.subcore_barrier()

    @pl.loop(0, PC)
    def _chunks(ci):
        base = (c * P_PAD + s * (PC * CB) + ci * CB) * G
        pltpu.sync_copy(xflat.at[pl.ds(base, G * CB)], rowbuf)
        pltpu.sync_copy(px5.at[s, ci], pxbuf)
        pltpu.sync_copy(q3.at[s, ci], wrow)
        _scale_rows(rowbuf, wrow, CB, qrowbuf=qrowbuf, zrow=zrow)
        for g in range(G):
            pltpu.sync_copy(rowbuf.at[pl.ds(g * CB, CB)],
                            acc.at[pxbuf.at[g]], add=True)
        pltpu.sync_copy(qrowbuf, cacc.at[pixbuf.at[ci]], add=True)

    plsc.subcore_barrier()
    r0x = s * STX
    pltpu.sync_copy(acc.at[pl.ds(r0x, STX)],
                    hsum.at[pl.ds(c * NSUPX + r0x, STX)])

    @pl.when(s == 0)
    def _rem0():
        pltpu.sync_copy(acc.at[pl.ds(NS * STX, REM * G)],
                        hsum.at[pl.ds(c * NSUPX + NS * STX, REM * G)])

    @pl.when(c == 0)
    def _():
        r0 = s * ST
        pltpu.sync_copy(cacc.at[pl.ds(r0, ST)], cnt.at[pl.ds(r0, ST)])

        @pl.when(s == 0)
        def _rem1():
            pltpu.sync_copy(cacc.at[pl.ds(REM_BASE, REM)],
                            cnt.at[pl.ds(REM_BASE, REM)])


@functools.cache
def _pool_kernel():
    return pl.kernel(
        _pool_body,
        out_type=(jax.ShapeDtypeStruct((NC * NSUPX, 16), jnp.float32),
                  jax.ShapeDtypeStruct((N_SUP, 16), jnp.float32)),
        mesh=_mesh(),
        scratch_types=[
            pltpu.VMEM_SHARED((NSUPX, 16), jnp.float32),
            pltpu.VMEM_SHARED((N_SUP, 16), jnp.float32),
            pltpu.VMEM((PC, CB), jnp.int32),
            pltpu.VMEM((G, CB), jnp.int32),
            pltpu.VMEM((CB,), jnp.float32),
            pltpu.VMEM((G * CB, 16), jnp.float32),
            pltpu.VMEM((CB, 16), jnp.float32),
            pltpu.VMEM((16,), jnp.float32),
            pltpu.SemaphoreType.DMA,
        ],
        **_SC_PARAMS,
    )


def _pool(*args):
    return _pool_kernel()(*args)


def _final_body(hx, pg6, q3, out0, out1, pgbuf, wrow, rowbuf, sem):
    c = lax.axis_index("c")
    s = lax.axis_index("s")
    nfull = jnp.where(s == NS - 1, 31, PC)

    def _do_chunk(ci, n):
        pltpu.sync_copy(pg6.at[c, s, ci], pgbuf)
        pltpu.sync_copy(q3.at[s, ci], wrow)
        for g in range(G):
            pltpu.async_copy(hx.at[pgbuf.at[g]],
                             rowbuf.at[pl.ds(g * CB, CB)], sem).wait()
        _scale_rows(rowbuf, wrow, CB)
        gbase = (s * (PC * CB) + ci * CB) * G
        for co, outr in ((0, out0), (1, out1)):
            @pl.when(c == co)
            def _():
                pltpu.sync_copy(rowbuf.at[pl.ds(0, n * G)],
                                outr.at[pl.ds(gbase, n * G)])

    @pl.loop(0, nfull)
    def _chunks(ci):
        _do_chunk(ci, CB)

    @pl.when(s == NS - 1)
    def _tail():
        _do_chunk(31, 32)


@functools.cache
def _final_kernel():
    return pl.kernel(
        _final_body,
        out_type=(jax.ShapeDtypeStruct((N_PIX * G, 16), jnp.float32),
                  jax.ShapeDtypeStruct((N_PIX * G, 16), jnp.float32)),
        mesh=_mesh(),
        scratch_types=[
            pltpu.VMEM((G, CB), jnp.int32),
            pltpu.VMEM((CB,), jnp.float32),
            pltpu.VMEM((G * CB, 16), jnp.float32),
            pltpu.SemaphoreType.DMA,
        ],
        **_SC_PARAMS,
    )


def _final(*args):
    return _final_kernel()(*args)


def _combine_mm_body(hsum_ref, cnt_ref, w_ref, b_ref, u_ref, y_ref):
    cntc = cnt_ref[:, 0:1]
    inv = jnp.where(cntc > 0, 1.0 / cntc, 0.0)
    h = hsum_ref[...] * inv
    y = jnp.dot(h, w_ref[...], preferred_element_type=jnp.float32,
                precision=lax.Precision.HIGHEST) + b_ref[...][None, :]
    u_ref[...] = h + y
    y_ref[...] = y


_TC_PARAMS = pltpu.CompilerParams(vmem_limit_bytes=100 * 1024 * 1024)

_combine_mm = pl.pallas_call(
    _combine_mm_body,
    out_shape=(jax.ShapeDtypeStruct((N_SUP, D), jnp.float32),
               jax.ShapeDtypeStruct((N_SUP, D), jnp.float32)),
    compiler_params=_TC_PARAMS,
)


def _bn(x, gamma, beta):
    mean = jnp.mean(x, axis=0, keepdims=True)
    var = jnp.mean((x - mean) ** 2, axis=0, keepdims=True)
    xn = (x - mean) * lax.rsqrt(var + EPS) * gamma[None, :] + beta[None, :]
    return jnp.where(xn >= 0, xn, 0.01 * xn)


def _bn_mm_body(u_ref, yprev_ref, g_ref, be_ref, w_ref, b_ref, un_ref, y_ref):
    h = _bn(u_ref[...] - yprev_ref[...], g_ref[...], be_ref[...])
    y = jnp.dot(h, w_ref[...], preferred_element_type=jnp.float32,
                precision=lax.Precision.HIGHEST) + b_ref[...][None, :]
    un_ref[...] = h + y
    y_ref[...] = y


_bn_mm = pl.pallas_call(
    _bn_mm_body,
    out_shape=(jax.ShapeDtypeStruct((N_SUP, D), jnp.float32),
               jax.ShapeDtypeStruct((N_SUP, D), jnp.float32)),
    compiler_params=_TC_PARAMS,
)


def _bn_only_body(u_ref, yprev_ref, g_ref, be_ref, h_ref):
    h_ref[...] = _bn(u_ref[...] - yprev_ref[...], g_ref[...], be_ref[...])


_bn_only = pl.pallas_call(
    _bn_only_body,
    out_shape=jax.ShapeDtypeStruct((N_SUP, D), jnp.float32),
    compiler_params=_TC_PARAMS,
)


def _expand4(idx, core_offset):
    base = (idx + core_offset) * G
    return jnp.reshape(base[:, None] + jnp.arange(G, dtype=jnp.int32), (-1,))


def _to_x(a2d):
    return a2d.reshape(-1, 16)


def _from_x(ax):
    return ax.reshape(-1, H)


def kernel(x, pix2sup, edge_index, edge_value, q_values, W, b, gamma, beta):
    src = edge_index[0].astype(jnp.int32)
    dst = edge_index[1].astype(jnp.int32)
    ew = edge_value.astype(jnp.float32)

    pad_e = E_PAD - E
    spread_e = (jnp.arange(pad_e, dtype=jnp.int32) * 7) % N_SUP
    srcp = jnp.concatenate([src, spread_e])
    dstp = jnp.concatenate([dst, spread_e])
    ewp = jnp.concatenate([ew, jnp.zeros((pad_e,), jnp.float32)])
    dx = _expand4(dstp, 0).reshape(NS, EC, IDXR, 128)
    sde = []
    for c in range(NC):
        sx = _expand4(srcp, c * N_SUP).reshape(NS, EC, IDXR, 128)
        sde.append(jnp.concatenate([sx, dx], axis=2))
    sd6 = jnp.stack(sde)
    ew3 = ewp.reshape(NS, EC, CBP)

    pad_p = P_PAD - N_PIX
    spread_p = (jnp.arange(pad_p, dtype=jnp.int32) * 11) % N_SUP
    pixp = jnp.concatenate([pix2sup.astype(jnp.int32), spread_p])
    qp = jnp.concatenate([q_values.astype(jnp.float32),
                          jnp.zeros((pad_p,), jnp.float32)])
    pix3 = pixp.reshape(NS, PC, CB)
    px5 = _expand4(pixp, 0).reshape(NS, PC, G, CB)
    pg6 = jnp.stack([_expand4(pixp, c * N_SUP).reshape(NS, PC, G, CB)
                     for c in range(NC)])
    q3 = qp.reshape(NS, PC, CB)

    xh = x.reshape(N_PIX, NC, H).transpose(1, 0, 2)
    xflat = _to_x(jnp.concatenate(
        [xh, jnp.zeros((NC, pad_p, H), jnp.float32)], axis=1).reshape(-1, H))

    hsumx, cnt = _pool(xflat, pix3, px5, q3)

    def _fold(flat):
        return jnp.concatenate([flat[:N_SUP], flat[N_SUP:]], axis=1)

    def _unfold(m):
        return jnp.concatenate([m[:, :H], m[:, H:]], axis=0)

    u_m, y_m = _combine_mm(_fold(_from_x(hsumx)), cnt, W[0], b[0])
    for i in range(NL):
        ux = _to_x(_unfold(u_m))
        yx = _to_x(_unfold(y_m))
        for _ in range(RHP):
            ux = _prop_update(ux, sd6, ew3, yx)
        u5_m = _fold(_from_x(ux))
        if i < NL - 1:
            u_m, y_m = _bn_mm(u5_m, y_m, gamma[i], beta[i],
                              W[i + 1], b[i + 1])
        else:
            h_m = _bn_only(u5_m, y_m, gamma[i], beta[i])
    out0, out1 = _final(_to_x(_unfold(h_m)), pg6, q3)
    return jnp.concatenate([_from_x(out0), _from_x(out1)], axis=1)
